# baseline (speedup 1.0000x reference)
import functools

import jax
import jax.numpy as jnp
from jax import lax
from jax.experimental import pallas as pl
from jax.experimental.pallas import tpu as pltpu
from jax.experimental.pallas import tpu_sc as plsc

N = 10000
E = 320000
D_IN = 192
H = 480
S0 = 128
B = 16
L = 3
D_OUT = 512

HP = 512
HC = 128
NCHUNK = HP // HC
NTILES = 32
EPT = E // NTILES
KE = 40
NBLK = EPT // KE
KEG = 80
NBLKG = EPT // KEG
NP = 10240
ROWS_PT = NP // 16

_MESH = plsc.VectorSubcoreMesh(core_axis_name="c", subcore_axis_name="s")


def _silu(z):
    return z / (1.0 + jnp.exp(-z))


def _silu_sc(z):
    t = z * (-1.4426950408889634)
    t = jnp.minimum(jnp.maximum(t, -126.0), 126.0)
    n = (t + 12582912.0) - 12582912.0
    f = t - n
    poly = jnp.full((16,), 0.000154035303933816, jnp.float32)
    for cc in (0.0013333558146428443, 0.009618129107628477,
               0.05550410866482158, 0.2402265069591007,
               0.6931471805599453, 1.0):
        poly = poly * f + cc
    ni = n.astype(jnp.int32)
    e2n = lax.bitcast_convert_type((ni + 127) << 23, jnp.float32)
    d = 1.0 + poly * e2n
    y = lax.bitcast_convert_type(
        jnp.full((16,), 0x7EF311C3, jnp.int32)
        - lax.bitcast_convert_type(d, jnp.int32), jnp.float32)
    y = y * (2.0 - d * y)
    y = y * (2.0 - d * y)
    y = y * (2.0 - d * y)
    return z * y


def _take16(v, i):
    idx = jnp.full((16, 1), i, jnp.int32)
    dnums = lax.GatherDimensionNumbers(
        offset_dims=(), collapsed_slice_dims=(0,), start_index_map=(0,))
    return lax.gather(v, idx, dnums, (1,),
                      mode=lax.GatherScatterMode.PROMISE_IN_BOUNDS)


@functools.partial(
    pl.kernel,
    out_type=tuple(jax.ShapeDtypeStruct((E,), jnp.float32) for _ in range(4)),
    mesh=_MESH,
    scratch_types=[
        pltpu.VMEM((KEG,), jnp.int32),
        pltpu.VMEM((KEG,), jnp.int32),
        tuple(pltpu.VMEM((KEG,), jnp.float32) for _ in range(6)),
        tuple(pltpu.VMEM((KEG,), jnp.float32) for _ in range(4)),
        pltpu.SemaphoreType.DMA,
        pltpu.SemaphoreType.DMA,
    ],
)
def _sc_geom(px, py, pz, row, col, rx, ry, rz, d2, row_v, col_v, g_v, o_v,
             sem, wsem):
    wid = lax.axis_index("c") * 16 + lax.axis_index("s")
    outs = (rx, ry, rz, d2)

    def block(bb, _):
        base = wid * EPT + bb * KEG
        ir = pltpu.async_copy(row.at[pl.ds(base, KEG)], row_v, sem)
        ic = pltpu.async_copy(col.at[pl.ds(base, KEG)], col_v, sem)
        ir.wait()
        ic.wait()
        tabs = (px, py, pz)
        gs = []
        for t in range(3):
            gs.append(pltpu.async_copy(tabs[t].at[row_v], g_v[t], sem))
            gs.append(pltpu.async_copy(tabs[t].at[col_v], g_v[3 + t], sem))
        for g in gs:
            g.wait()
        prx, pry, prz, pcx, pcy, pcz = g_v
        for g in range(KEG // 16):
            sl = pl.ds(g * 16, 16)
            dx = prx[sl] - pcx[sl]
            dy = pry[sl] - pcy[sl]
            dz = prz[sl] - pcz[sl]
            o_v[0][sl] = dx
            o_v[1][sl] = dy
            o_v[2][sl] = dz
            o_v[3][sl] = dx * dx + dy * dy + dz * dz
        ws = [pltpu.async_copy(o_v[t], outs[t].at[pl.ds(base, KEG)], wsem)
              for t in range(4)]
        for w in ws:
            w.wait()
        return 0

    lax.fori_loop(0, NBLKG, block, 0)


SG = 2000
NSB = SG // 80
NST = EPT // SG


@functools.partial(
    pl.kernel,
    out_type=jax.ShapeDtypeStruct((2, NP, HC), jnp.float32),
    mesh=_MESH,
    scratch_types=[
        pltpu.VMEM_SHARED((NP, HC), jnp.float32),
        pltpu.VMEM((SG,), jnp.int32),
        pltpu.VMEM((SG,), jnp.int32),
        pltpu.VMEM((80,), jnp.int32),
        pltpu.VMEM((4 * SG,), jnp.float32),
        tuple(pltpu.VMEM((KE, HC), jnp.float32) for _ in range(2)),
        tuple(pltpu.VMEM((KE, HC), jnp.float32) for _ in range(2)),
        pltpu.VMEM((80, HC), jnp.float32),
        pltpu.VMEM((4, HC), jnp.float32),
        tuple(pltpu.SemaphoreType.DMA for _ in range(2)),
        tuple(pltpu.SemaphoreType.DMA for _ in range(2)),
        pltpu.SemaphoreType.DMA,
    ],
)
def _sc_edge(ha, hb, ea_flat, row, col, we, part, agg_sh, srow, scol, cols_v,
             sea, pr_v, pc_v, z_v, we_v, a_sem, b_sem, c_sem):
    c = lax.axis_index("c")
    s = lax.axis_index("s")
    wid = c * 16 + s
    zero16 = jnp.zeros((16,), jnp.float32)

    def zrow(r, _):
        for j in range(HC // 16):
            z_v[r, pl.ds(j * 16, 16)] = zero16
        return 0

    lax.fori_loop(0, 80, zrow, 0)
    for i in range(ROWS_PT // 80):
        pltpu.sync_copy(z_v, agg_sh.at[pl.ds(s * ROWS_PT + i * 80, 80)])

    pltpu.sync_copy(we, we_v)
    plsc.subcore_barrier()

    def start_g(sb, hh):
        off = sb * 80 + hh * KE
        pltpu.async_copy(ha.at[srow.at[pl.ds(off, KE)]], pr_v[hh], a_sem[hh])
        pltpu.async_copy(hb.at[scol.at[pl.ds(off, KE)]], pc_v[hh], b_sem[hh])

    def wait_g(hh):
        pltpu.make_async_copy(ha.at[srow.at[pl.ds(0, KE)]], pr_v[hh],
                              a_sem[hh]).wait()
        pltpu.make_async_copy(hb.at[scol.at[pl.ds(0, KE)]], pc_v[hh],
                              b_sem[hh]).wait()

    def stage(st, _):
        sbase = wid * EPT + st * SG
        pltpu.sync_copy(row.at[pl.ds(sbase, SG)], srow)
        pltpu.sync_copy(col.at[pl.ds(sbase, SG)], scol)
        pltpu.sync_copy(ea_flat.at[pl.ds(sbase * 4, 4 * SG)], sea)
        start_g(0, 0)
        start_g(0, 1)

        def sblock(sb, _):
            pltpu.async_copy(col.at[pl.ds(sbase + sb * 80, 80)], cols_v,
                             c_sem)
            for hh in range(2):
                wait_g(hh)
                prp, pcp = pr_v[hh], pc_v[hh]

                @plsc.parallel_loop(0, KE // 4, unroll=2)
                def _grp(g):
                    coff = sb * 320 + hh * 160 + g * 16
                    cvec = sea[pl.ds(coff, 16)]
                    for kk in range(4):
                        k = g * 4 + kk
                        a = [_take16(cvec, kk * 4 + j) for j in range(4)]
                        for ch in range(HC // 16):
                            sl = pl.ds(ch * 16, 16)
                            ew = (a[0] * we_v[0, sl] + a[1] * we_v[1, sl]
                                  + a[2] * we_v[2, sl] + a[3] * we_v[3, sl])
                            z = prp[k, sl] * pcp[k, sl] * ew
                            z_v[hh * KE + k, sl] = _silu_sc(z)

                @pl.when(sb + 1 < NSB)
                def _():
                    start_g(sb + 1, hh)

            pltpu.make_async_copy(col.at[pl.ds(0, 80)], cols_v, c_sem).wait()
            pltpu.sync_copy(z_v, agg_sh.at[cols_v], add=True)
            return 0

        lax.fori_loop(0, NSB, sblock, 0)
        return 0

    lax.fori_loop(0, NST, stage, 0)

    plsc.subcore_barrier()
    pltpu.sync_copy(agg_sh.at[pl.ds(s * ROWS_PT, ROWS_PT)],
                    part.at[c, pl.ds(s * ROWS_PT, ROWS_PT)])


_MB = 2000


def _tc_proj(x, w0):
    def body(x_ref, w_ref, o_ref):
        o_ref[...] = jnp.dot(x_ref[...], w_ref[...],
                             preferred_element_type=jnp.float32)

    return pl.pallas_call(
        body,
        grid=(N // _MB,),
        in_specs=[pl.BlockSpec((_MB, D_IN), lambda i: (i, 0)),
                  pl.BlockSpec((D_IN, H), lambda i: (0, 0))],
        out_specs=pl.BlockSpec((_MB, H), lambda i: (i, 0)),
        out_shape=jax.ShapeDtypeStruct((N, H), jnp.float32),
    )(x, w0)


def _tc_pre(h, wa, wb):
    def body(h_ref, wa_ref, wb_ref, a_ref, b_ref):
        hv = h_ref[...]
        a_ref[...] = jnp.dot(hv, wa_ref[...], preferred_element_type=jnp.float32)
        b_ref[...] = jnp.dot(hv, wb_ref[...], preferred_element_type=jnp.float32)

    return pl.pallas_call(
        body,
        grid=(N // _MB,),
        in_specs=[pl.BlockSpec((_MB, H), lambda i: (i, 0)),
                  pl.BlockSpec((H, HP), lambda i: (0, 0)),
                  pl.BlockSpec((H, HP), lambda i: (0, 0))],
        out_specs=[pl.BlockSpec((_MB, HP), lambda i: (i, 0)),
                   pl.BlockSpec((_MB, HP), lambda i: (i, 0))],
        out_shape=[jax.ShapeDtypeStruct((N, HP), jnp.float32),
                   jax.ShapeDtypeStruct((N, HP), jnp.float32)],
    )(h, wa, wb)


def _tc_post(parts, h, wm, wc, wd, wu):
    def body(p0_ref, p1_ref, p2_ref, p3_ref, h_ref, wm_ref, wc_ref, wd_ref,
             wu_ref, o_ref):
        agg = jnp.zeros((_MB, H), jnp.float32)
        for j, p_ref in enumerate((p0_ref, p1_ref, p2_ref, p3_ref)):
            pj = p_ref[0] + p_ref[1]
            agg = agg + jnp.dot(pj, wm_ref[pl.ds(j * HC, HC), :],
                                preferred_element_type=jnp.float32)
        hv = h_ref[...]
        u = (jnp.dot(hv, wc_ref[...], preferred_element_type=jnp.float32)
             * jnp.dot(agg, wd_ref[...], preferred_element_type=jnp.float32))
        o_ref[...] = jnp.dot(_silu(u), wu_ref[...],
                             preferred_element_type=jnp.float32)

    pspec = pl.BlockSpec((2, _MB, HC), lambda i: (0, i, 0))
    wspec = pl.BlockSpec((H, H), lambda i: (0, 0))
    return pl.pallas_call(
        body,
        grid=(N // _MB,),
        in_specs=[pspec, pspec, pspec, pspec,
                  pl.BlockSpec((_MB, H), lambda i: (i, 0)),
                  pl.BlockSpec((HP, H), lambda i: (0, 0)),
                  wspec, wspec, wspec],
        out_specs=pl.BlockSpec((_MB, H), lambda i: (i, 0)),
        out_shape=jax.ShapeDtypeStruct((N, H), jnp.float32),
    )(*parts, h, wm, wc, wd, wu)


def _tc_norm(rx, ry, rz, d2):
    def body(rx_ref, ry_ref, rz_ref, d2_ref, nx_ref, ny_ref, nz_ref, dd_ref):
        dist = jnp.sqrt(d2_ref[...])
        inv = 1.0 / (dist + 1e-08)
        nx_ref[...] = rx_ref[...] * inv
        ny_ref[...] = ry_ref[...] * inv
        nz_ref[...] = rz_ref[...] * inv
        dd_ref[...] = dist

    shp = jax.ShapeDtypeStruct((E // 128, 128), jnp.float32)
    return pl.pallas_call(body, out_shape=[shp] * 4)(rx, ry, rz, d2)


def _tc_pool(h, batch2d, wout):
    def body(h_ref, b_ref, w_ref, o_ref, sum_acc, cnt_acc):
        i = pl.program_id(0)

        @pl.when(i == 0)
        def _():
            sum_acc[...] = jnp.zeros((B, S0), jnp.float32)
            cnt_acc[...] = jnp.zeros((1, B), jnp.float32)

        inv = h_ref[:, pl.ds(0, S0)]
        oh = (b_ref[...] == lax.broadcasted_iota(jnp.int32, (_MB, B), 1)
              ).astype(jnp.float32)
        sum_acc[...] += lax.dot_general(oh, inv, (((0,), (0,)), ((), ())),
                                        preferred_element_type=jnp.float32)
        cnt_acc[...] += jnp.sum(oh, axis=0, keepdims=True)

        @pl.when(i == N // _MB - 1)
        def _():
            pooled = sum_acc[...] / jnp.maximum(cnt_acc[...], 1.0).reshape(B, 1)
            o_ref[...] = jnp.dot(pooled, w_ref[...],
                                 preferred_element_type=jnp.float32)

    return pl.pallas_call(
        body,
        grid=(N // _MB,),
        in_specs=[pl.BlockSpec((_MB, H), lambda i: (i, 0)),
                  pl.BlockSpec((_MB, 1), lambda i: (i, 0)),
                  pl.BlockSpec((S0, D_OUT), lambda i: (0, 0))],
        out_specs=pl.BlockSpec((B, D_OUT), lambda i: (0, 0)),
        out_shape=jax.ShapeDtypeStruct((B, D_OUT), jnp.float32),
        scratch_shapes=[pltpu.VMEM((B, S0), jnp.float32),
                        pltpu.VMEM((1, B), jnp.float32)],
    )(h, batch2d, wout)


def kernel(x, pos, edge_index, batch, W0, Wa, Wb, We, Wm, Wc, Wd, Wu, Wout):
    row, col = edge_index[0], edge_index[1]

    rx, ry, rz, d2 = _sc_geom(pos[:, 0], pos[:, 1], pos[:, 2], row, col)
    nx, ny, nz, dd = _tc_norm(rx.reshape(E // 128, 128),
                              ry.reshape(E // 128, 128),
                              rz.reshape(E // 128, 128),
                              d2.reshape(E // 128, 128))
    ea_flat = jnp.stack([nx.reshape(E), ny.reshape(E), nz.reshape(E),
                         dd.reshape(E)], axis=1).reshape(4 * E)

    pad = ((0, 0), (0, 0), (0, HP - H))
    Wa_p = jnp.pad(Wa, pad)
    Wb_p = jnp.pad(Wb, pad)
    We_p = jnp.pad(We, pad)
    Wm_p = jnp.pad(Wm, ((0, 0), (0, HP - H), (0, 0)))

    h = _tc_proj(x, W0)
    for l in range(L):
        ha, hb = _tc_pre(h, Wa_p[l], Wb_p[l])
        parts = []
        for j in range(NCHUNK):
            sl = slice(j * HC, (j + 1) * HC)
            parts.append(_sc_edge(ha[:, sl], hb[:, sl], ea_flat, row, col,
                                  We_p[l][:, sl]))
        h = _tc_post(parts, h, Wm_p[l], Wc[l], Wd[l], Wu[l])

    return _tc_pool(h, batch.reshape(N, 1), Wout)


# speedup vs baseline: 1.6629x; 1.6629x over previous
import functools

import jax
import jax.numpy as jnp
from jax import lax
from jax.experimental import pallas as pl
from jax.experimental.pallas import tpu as pltpu
from jax.experimental.pallas import tpu_sc as plsc

N = 10000
E = 320000
D_IN = 192
H = 480
S0 = 128
B = 16
L = 3
D_OUT = 512

HP = 512
HC = 128
NCHUNK = HP // HC
NTILES = 32
EPT = E // NTILES
KE = 40
NBLK = EPT // KE
KEG = 80
NBLKG = EPT // KEG
NP = 10240
ROWS_PT = NP // 16

_MESH = plsc.VectorSubcoreMesh(core_axis_name="c", subcore_axis_name="s")


def _silu(z):
    return z / (1.0 + jnp.exp(-z))


def _silu_sc(z):
    t = z * (-1.4426950408889634)
    t = jnp.minimum(jnp.maximum(t, -126.0), 126.0)
    n = (t + 12582912.0) - 12582912.0
    f = t - n
    poly = jnp.full((16,), 0.000154035303933816, jnp.float32)
    for cc in (0.0013333558146428443, 0.009618129107628477,
               0.05550410866482158, 0.2402265069591007,
               0.6931471805599453, 1.0):
        poly = poly * f + cc
    ni = n.astype(jnp.int32)
    e2n = lax.bitcast_convert_type((ni + 127) << 23, jnp.float32)
    d = 1.0 + poly * e2n
    y = lax.bitcast_convert_type(
        jnp.full((16,), 0x7EF311C3, jnp.int32)
        - lax.bitcast_convert_type(d, jnp.int32), jnp.float32)
    y = y * (2.0 - d * y)
    y = y * (2.0 - d * y)
    y = y * (2.0 - d * y)
    return z * y


def _take16(v, i):
    idx = jnp.full((16, 1), i, jnp.int32)
    dnums = lax.GatherDimensionNumbers(
        offset_dims=(), collapsed_slice_dims=(0,), start_index_map=(0,))
    return lax.gather(v, idx, dnums, (1,),
                      mode=lax.GatherScatterMode.PROMISE_IN_BOUNDS)


@functools.partial(
    pl.kernel,
    out_type=tuple(jax.ShapeDtypeStruct((E,), jnp.float32) for _ in range(4)),
    mesh=_MESH,
    scratch_types=[
        pltpu.VMEM((KEG,), jnp.int32),
        pltpu.VMEM((KEG,), jnp.int32),
        tuple(pltpu.VMEM((KEG,), jnp.float32) for _ in range(6)),
        tuple(pltpu.VMEM((KEG,), jnp.float32) for _ in range(4)),
        pltpu.SemaphoreType.DMA,
        pltpu.SemaphoreType.DMA,
    ],
)
def _sc_geom(px, py, pz, row, col, rx, ry, rz, d2, row_v, col_v, g_v, o_v,
             sem, wsem):
    wid = lax.axis_index("c") * 16 + lax.axis_index("s")
    outs = (rx, ry, rz, d2)

    def block(bb, _):
        base = wid * EPT + bb * KEG
        ir = pltpu.async_copy(row.at[pl.ds(base, KEG)], row_v, sem)
        ic = pltpu.async_copy(col.at[pl.ds(base, KEG)], col_v, sem)
        ir.wait()
        ic.wait()
        tabs = (px, py, pz)
        gs = []
        for t in range(3):
            gs.append(pltpu.async_copy(tabs[t].at[row_v], g_v[t], sem))
            gs.append(pltpu.async_copy(tabs[t].at[col_v], g_v[3 + t], sem))
        for g in gs:
            g.wait()
        prx, pry, prz, pcx, pcy, pcz = g_v
        for g in range(KEG // 16):
            sl = pl.ds(g * 16, 16)
            dx = prx[sl] - pcx[sl]
            dy = pry[sl] - pcy[sl]
            dz = prz[sl] - pcz[sl]
            o_v[0][sl] = dx
            o_v[1][sl] = dy
            o_v[2][sl] = dz
            o_v[3][sl] = dx * dx + dy * dy + dz * dz
        ws = [pltpu.async_copy(o_v[t], outs[t].at[pl.ds(base, KEG)], wsem)
              for t in range(4)]
        for w in ws:
            w.wait()
        return 0

    lax.fori_loop(0, NBLKG, block, 0)


SG = 2000
NSB = SG // 80
NST = EPT // SG


@functools.partial(
    pl.kernel,
    out_type=jax.ShapeDtypeStruct((2, NP, HC), jnp.float32),
    mesh=_MESH,
    scratch_types=[
        pltpu.VMEM_SHARED((NP, HC), jnp.float32),
        pltpu.VMEM((SG,), jnp.int32),
        pltpu.VMEM((SG,), jnp.int32),
        pltpu.VMEM((80,), jnp.int32),
        pltpu.VMEM((4 * SG,), jnp.float32),
        tuple(pltpu.VMEM((KE, HC), jnp.float32) for _ in range(2)),
        tuple(pltpu.VMEM((KE, HC), jnp.float32) for _ in range(2)),
        pltpu.VMEM((80, HC), jnp.float32),
        pltpu.VMEM((4, HC), jnp.float32),
        tuple(pltpu.SemaphoreType.DMA for _ in range(2)),
        tuple(pltpu.SemaphoreType.DMA for _ in range(2)),
        pltpu.SemaphoreType.DMA,
    ],
)
def _sc_edge(ha, hb, ea_flat, row, col, we, part, agg_sh, srow, scol, cols_v,
             sea, pr_v, pc_v, z_v, we_v, a_sem, b_sem, c_sem):
    c = lax.axis_index("c")
    s = lax.axis_index("s")
    wid = c * 16 + s
    zero16 = jnp.zeros((16,), jnp.float32)

    def zrow(r, _):
        for j in range(HC // 16):
            z_v[r, pl.ds(j * 16, 16)] = zero16
        return 0

    lax.fori_loop(0, 80, zrow, 0)
    for i in range(ROWS_PT // 80):
        pltpu.sync_copy(z_v, agg_sh.at[pl.ds(s * ROWS_PT + i * 80, 80)])

    pltpu.sync_copy(we, we_v)
    plsc.subcore_barrier()

    def start_g(sb, hh):
        off = sb * 80 + hh * KE
        pltpu.async_copy(ha.at[srow.at[pl.ds(off, KE)]], pr_v[hh], a_sem[hh])
        pltpu.async_copy(hb.at[scol.at[pl.ds(off, KE)]], pc_v[hh], b_sem[hh])

    def wait_g(hh):
        pltpu.make_async_copy(ha.at[srow.at[pl.ds(0, KE)]], pr_v[hh],
                              a_sem[hh]).wait()
        pltpu.make_async_copy(hb.at[scol.at[pl.ds(0, KE)]], pc_v[hh],
                              b_sem[hh]).wait()

    def stage(st, _):
        sbase = wid * EPT + st * SG
        pltpu.sync_copy(row.at[pl.ds(sbase, SG)], srow)
        pltpu.sync_copy(col.at[pl.ds(sbase, SG)], scol)
        pltpu.sync_copy(ea_flat.at[pl.ds(sbase * 4, 4 * SG)], sea)
        start_g(0, 0)
        start_g(0, 1)

        def sblock(sb, _):
            pltpu.async_copy(col.at[pl.ds(sbase + sb * 80, 80)], cols_v,
                             c_sem)
            for hh in range(2):
                wait_g(hh)
                prp, pcp = pr_v[hh], pc_v[hh]

                @plsc.parallel_loop(0, KE // 4, unroll=1)
                def _grp(g):
                    coff = sb * 320 + hh * 160 + g * 16
                    cvec = sea[pl.ds(coff, 16)]
                    for kk in range(4):
                        k = g * 4 + kk
                        a = [_take16(cvec, kk * 4 + j) for j in range(4)]
                        for ch in range(HC // 16):
                            sl = pl.ds(ch * 16, 16)
                            ew = (a[0] * we_v[0, sl] + a[1] * we_v[1, sl]
                                  + a[2] * we_v[2, sl] + a[3] * we_v[3, sl])
                            z = prp[k, sl] * pcp[k, sl] * ew
                            z_v[hh * KE + k, sl] = _silu_sc(z)

                @pl.when(sb + 1 < NSB)
                def _():
                    start_g(sb + 1, hh)

            pltpu.make_async_copy(col.at[pl.ds(0, 80)], cols_v, c_sem).wait()
            pltpu.sync_copy(z_v, agg_sh.at[cols_v], add=True)
            return 0

        lax.fori_loop(0, NSB, sblock, 0)
        return 0

    lax.fori_loop(0, NST, stage, 0)

    plsc.subcore_barrier()
    pltpu.sync_copy(agg_sh.at[pl.ds(s * ROWS_PT, ROWS_PT)],
                    part.at[c, pl.ds(s * ROWS_PT, ROWS_PT)])


_MB = 2000


def _tc_proj(x, w0):
    def body(x_ref, w_ref, o_ref):
        o_ref[...] = jnp.dot(x_ref[...], w_ref[...],
                             preferred_element_type=jnp.float32)

    return pl.pallas_call(
        body,
        grid=(N // _MB,),
        in_specs=[pl.BlockSpec((_MB, D_IN), lambda i: (i, 0)),
                  pl.BlockSpec((D_IN, H), lambda i: (0, 0))],
        out_specs=pl.BlockSpec((_MB, H), lambda i: (i, 0)),
        out_shape=jax.ShapeDtypeStruct((N, H), jnp.float32),
    )(x, w0)


def _tc_pre(h, wa, wb):
    def body(h_ref, wa_ref, wb_ref, a_ref, b_ref):
        hv = h_ref[...]
        a_ref[...] = jnp.dot(hv, wa_ref[...], preferred_element_type=jnp.float32)
        b_ref[...] = jnp.dot(hv, wb_ref[...], preferred_element_type=jnp.float32)

    return pl.pallas_call(
        body,
        grid=(N // _MB,),
        in_specs=[pl.BlockSpec((_MB, H), lambda i: (i, 0)),
                  pl.BlockSpec((H, HP), lambda i: (0, 0)),
                  pl.BlockSpec((H, HP), lambda i: (0, 0))],
        out_specs=[pl.BlockSpec((_MB, HP), lambda i: (i, 0)),
                   pl.BlockSpec((_MB, HP), lambda i: (i, 0))],
        out_shape=[jax.ShapeDtypeStruct((N, HP), jnp.float32),
                   jax.ShapeDtypeStruct((N, HP), jnp.float32)],
    )(h, wa, wb)


def _tc_post(parts, h, wm, wc, wd, wu):
    def body(p0_ref, p1_ref, p2_ref, p3_ref, h_ref, wm_ref, wc_ref, wd_ref,
             wu_ref, o_ref):
        agg = jnp.zeros((_MB, H), jnp.float32)
        for j, p_ref in enumerate((p0_ref, p1_ref, p2_ref, p3_ref)):
            pj = p_ref[0] + p_ref[1]
            agg = agg + jnp.dot(pj, wm_ref[pl.ds(j * HC, HC), :],
                                preferred_element_type=jnp.float32)
        hv = h_ref[...]
        u = (jnp.dot(hv, wc_ref[...], preferred_element_type=jnp.float32)
             * jnp.dot(agg, wd_ref[...], preferred_element_type=jnp.float32))
        o_ref[...] = jnp.dot(_silu(u), wu_ref[...],
                             preferred_element_type=jnp.float32)

    pspec = pl.BlockSpec((2, _MB, HC), lambda i: (0, i, 0))
    wspec = pl.BlockSpec((H, H), lambda i: (0, 0))
    return pl.pallas_call(
        body,
        grid=(N // _MB,),
        in_specs=[pspec, pspec, pspec, pspec,
                  pl.BlockSpec((_MB, H), lambda i: (i, 0)),
                  pl.BlockSpec((HP, H), lambda i: (0, 0)),
                  wspec, wspec, wspec],
        out_specs=pl.BlockSpec((_MB, H), lambda i: (i, 0)),
        out_shape=jax.ShapeDtypeStruct((N, H), jnp.float32),
    )(*parts, h, wm, wc, wd, wu)


def _tc_norm(rx, ry, rz, d2):
    def body(rx_ref, ry_ref, rz_ref, d2_ref, nx_ref, ny_ref, nz_ref, dd_ref):
        dist = jnp.sqrt(d2_ref[...])
        inv = 1.0 / (dist + 1e-08)
        nx_ref[...] = rx_ref[...] * inv
        ny_ref[...] = ry_ref[...] * inv
        nz_ref[...] = rz_ref[...] * inv
        dd_ref[...] = dist

    shp = jax.ShapeDtypeStruct((E // 128, 128), jnp.float32)
    return pl.pallas_call(body, out_shape=[shp] * 4)(rx, ry, rz, d2)


def _tc_pool(h, batch2d, wout):
    def body(h_ref, b_ref, w_ref, o_ref, sum_acc, cnt_acc):
        i = pl.program_id(0)

        @pl.when(i == 0)
        def _():
            sum_acc[...] = jnp.zeros((B, S0), jnp.float32)
            cnt_acc[...] = jnp.zeros((1, B), jnp.float32)

        inv = h_ref[:, pl.ds(0, S0)]
        oh = (b_ref[...] == lax.broadcasted_iota(jnp.int32, (_MB, B), 1)
              ).astype(jnp.float32)
        sum_acc[...] += lax.dot_general(oh, inv, (((0,), (0,)), ((), ())),
                                        preferred_element_type=jnp.float32)
        cnt_acc[...] += jnp.sum(oh, axis=0, keepdims=True)

        @pl.when(i == N // _MB - 1)
        def _():
            pooled = sum_acc[...] / jnp.maximum(cnt_acc[...], 1.0).reshape(B, 1)
            o_ref[...] = jnp.dot(pooled, w_ref[...],
                                 preferred_element_type=jnp.float32)

    return pl.pallas_call(
        body,
        grid=(N // _MB,),
        in_specs=[pl.BlockSpec((_MB, H), lambda i: (i, 0)),
                  pl.BlockSpec((_MB, 1), lambda i: (i, 0)),
                  pl.BlockSpec((S0, D_OUT), lambda i: (0, 0))],
        out_specs=pl.BlockSpec((B, D_OUT), lambda i: (0, 0)),
        out_shape=jax.ShapeDtypeStruct((B, D_OUT), jnp.float32),
        scratch_shapes=[pltpu.VMEM((B, S0), jnp.float32),
                        pltpu.VMEM((1, B), jnp.float32)],
    )(h, batch2d, wout)


def kernel(x, pos, edge_index, batch, W0, Wa, Wb, We, Wm, Wc, Wd, Wu, Wout):
    row, col = edge_index[0], edge_index[1]

    rx, ry, rz, d2 = _sc_geom(pos[:, 0], pos[:, 1], pos[:, 2], row, col)
    nx, ny, nz, dd = _tc_norm(rx.reshape(E // 128, 128),
                              ry.reshape(E // 128, 128),
                              rz.reshape(E // 128, 128),
                              d2.reshape(E // 128, 128))
    ea_flat = jnp.stack([nx.reshape(E), ny.reshape(E), nz.reshape(E),
                         dd.reshape(E)], axis=1).reshape(4 * E)

    pad = ((0, 0), (0, 0), (0, HP - H))
    Wa_p = jnp.pad(Wa, pad)
    Wb_p = jnp.pad(Wb, pad)
    We_p = jnp.pad(We, pad)
    Wm_p = jnp.pad(Wm, ((0, 0), (0, HP - H), (0, 0)))

    h = _tc_proj(x, W0)
    for l in range(L):
        ha, hb = _tc_pre(h, Wa_p[l], Wb_p[l])
        parts = []
        for j in range(NCHUNK):
            sl = slice(j * HC, (j + 1) * HC)
            parts.append(_sc_edge(ha[:, sl], hb[:, sl], ea_flat, row, col,
                                  We_p[l][:, sl]))
        h = _tc_post(parts, h, Wm_p[l], Wc[l], Wd[l], Wu[l])

    return _tc_pool(h, batch.reshape(N, 1), Wout)


# speedup vs baseline: 7.1371x; 4.2920x over previous
import functools

import jax
import jax.numpy as jnp
from jax import lax
from jax.experimental import pallas as pl
from jax.experimental.pallas import tpu as pltpu
from jax.experimental.pallas import tpu_sc as plsc

N = 10000
E = 320000
D_IN = 192
H = 480
S0 = 128
B = 16
L = 3
D_OUT = 512

HP = 512
HC = 128
NCHUNK = HP // HC
NTILES = 32
EPT = E // NTILES
KE = 40
NBLK = EPT // KE
KEG = 80
NBLKG = EPT // KEG
NP = 10240
ROWS_PT = NP // 16

_MESH = plsc.VectorSubcoreMesh(core_axis_name="c", subcore_axis_name="s")


def _silu(z):
    return z / (1.0 + jnp.exp(-z))


def _take16(v, i):
    idx = jnp.full((16, 1), i, jnp.int32)
    dnums = lax.GatherDimensionNumbers(
        offset_dims=(), collapsed_slice_dims=(0,), start_index_map=(0,))
    return lax.gather(v, idx, dnums, (1,),
                      mode=lax.GatherScatterMode.PROMISE_IN_BOUNDS)


@functools.partial(
    pl.kernel,
    out_type=tuple(jax.ShapeDtypeStruct((E,), jnp.float32) for _ in range(4)),
    mesh=_MESH,
    scratch_types=[
        pltpu.VMEM((KEG,), jnp.int32),
        pltpu.VMEM((KEG,), jnp.int32),
        tuple(pltpu.VMEM((KEG,), jnp.float32) for _ in range(6)),
        tuple(pltpu.VMEM((KEG,), jnp.float32) for _ in range(4)),
        pltpu.SemaphoreType.DMA,
        pltpu.SemaphoreType.DMA,
    ],
)
def _sc_geom(px, py, pz, row, col, rx, ry, rz, d2, row_v, col_v, g_v, o_v,
             sem, wsem):
    wid = lax.axis_index("c") * 16 + lax.axis_index("s")
    outs = (rx, ry, rz, d2)

    def block(bb, _):
        base = wid * EPT + bb * KEG
        ir = pltpu.async_copy(row.at[pl.ds(base, KEG)], row_v, sem)
        ic = pltpu.async_copy(col.at[pl.ds(base, KEG)], col_v, sem)
        ir.wait()
        ic.wait()
        tabs = (px, py, pz)
        gs = []
        for t in range(3):
            gs.append(pltpu.async_copy(tabs[t].at[row_v], g_v[t], sem))
            gs.append(pltpu.async_copy(tabs[t].at[col_v], g_v[3 + t], sem))
        for g in gs:
            g.wait()
        prx, pry, prz, pcx, pcy, pcz = g_v
        for g in range(KEG // 16):
            sl = pl.ds(g * 16, 16)
            dx = prx[sl] - pcx[sl]
            dy = pry[sl] - pcy[sl]
            dz = prz[sl] - pcz[sl]
            o_v[0][sl] = dx
            o_v[1][sl] = dy
            o_v[2][sl] = dz
            o_v[3][sl] = dx * dx + dy * dy + dz * dz
        ws = [pltpu.async_copy(o_v[t], outs[t].at[pl.ds(base, KEG)], wsem)
              for t in range(4)]
        for w in ws:
            w.wait()
        return 0

    lax.fori_loop(0, NBLKG, block, 0)


SG = 2000
NSB = SG // 80
NST = EPT // SG


@functools.partial(
    pl.kernel,
    out_type=jax.ShapeDtypeStruct((E, HC), jnp.float32),
    mesh=_MESH,
    scratch_types=[
        pltpu.VMEM((SG,), jnp.int32),
        pltpu.VMEM((SG,), jnp.int32),
        pltpu.VMEM((4 * SG,), jnp.float32),
        tuple(pltpu.VMEM((KE, HC), jnp.float32) for _ in range(2)),
        tuple(pltpu.VMEM((KE, HC), jnp.float32) for _ in range(2)),
        pltpu.VMEM((80, HC), jnp.float32),
        pltpu.VMEM((4, HC), jnp.float32),
        tuple(pltpu.SemaphoreType.DMA for _ in range(2)),
        tuple(pltpu.SemaphoreType.DMA for _ in range(2)),
    ],
)
def _sc_mul(ha, hb, ea_flat, row, col, we, zout, srow, scol, sea, pr_v, pc_v,
            z_v, we_v, a_sem, b_sem):
    c = lax.axis_index("c")
    s = lax.axis_index("s")
    wid = c * 16 + s
    pltpu.sync_copy(we, we_v)

    def start_g(sb, hh):
        off = sb * 80 + hh * KE
        pltpu.async_copy(ha.at[srow.at[pl.ds(off, KE)]], pr_v[hh], a_sem[hh])
        pltpu.async_copy(hb.at[scol.at[pl.ds(off, KE)]], pc_v[hh], b_sem[hh])

    def wait_g(hh):
        pltpu.make_async_copy(ha.at[srow.at[pl.ds(0, KE)]], pr_v[hh],
                              a_sem[hh]).wait()
        pltpu.make_async_copy(hb.at[scol.at[pl.ds(0, KE)]], pc_v[hh],
                              b_sem[hh]).wait()

    def stage(st, _):
        sbase = wid * EPT + st * SG
        pltpu.sync_copy(row.at[pl.ds(sbase, SG)], srow)
        pltpu.sync_copy(col.at[pl.ds(sbase, SG)], scol)
        pltpu.sync_copy(ea_flat.at[pl.ds(sbase * 4, 4 * SG)], sea)
        start_g(0, 0)
        start_g(0, 1)

        def sblock(sb, _):
            for hh in range(2):
                wait_g(hh)
                prp, pcp = pr_v[hh], pc_v[hh]

                @plsc.parallel_loop(0, KE // 4, unroll=2)
                def _grp(g):
                    coff = sb * 320 + hh * 160 + g * 16
                    cvec = sea[pl.ds(coff, 16)]
                    for kk in range(4):
                        k = g * 4 + kk
                        a = [_take16(cvec, kk * 4 + j) for j in range(4)]
                        for ch in range(HC // 16):
                            sl = pl.ds(ch * 16, 16)
                            ew = (a[0] * we_v[0, sl] + a[1] * we_v[1, sl]
                                  + a[2] * we_v[2, sl] + a[3] * we_v[3, sl])
                            z_v[hh * KE + k, sl] = (prp[k, sl] * pcp[k, sl]
                                                    * ew)

                @pl.when(sb + 1 < NSB)
                def _():
                    start_g(sb + 1, hh)

            pltpu.sync_copy(z_v, zout.at[pl.ds(sbase + sb * 80, 80)])
            return 0

        lax.fori_loop(0, NSB, sblock, 0)
        return 0

    lax.fori_loop(0, NST, stage, 0)


@functools.partial(
    pl.kernel,
    out_type=jax.ShapeDtypeStruct((2, NP, HC), jnp.float32),
    mesh=_MESH,
    scratch_types=[
        pltpu.VMEM_SHARED((NP, HC), jnp.float32),
        pltpu.VMEM((80, HC), jnp.float32),
        pltpu.VMEM((80,), jnp.int32),
        pltpu.SemaphoreType.DMA,
    ],
)
def _sc_scat(zsil, col, part, agg_sh, zl_v, cols_v, c_sem):
    c = lax.axis_index("c")
    s = lax.axis_index("s")
    wid = c * 16 + s
    zero16 = jnp.zeros((16,), jnp.float32)

    def zrow(r, _):
        for j in range(HC // 16):
            zl_v[r, pl.ds(j * 16, 16)] = zero16
        return 0

    lax.fori_loop(0, 80, zrow, 0)
    for i in range(ROWS_PT // 80):
        pltpu.sync_copy(zl_v, agg_sh.at[pl.ds(s * ROWS_PT + i * 80, 80)])
    plsc.subcore_barrier()

    def block(b, _):
        base = wid * EPT + b * 80
        cd = pltpu.async_copy(col.at[pl.ds(base, 80)], cols_v, c_sem)
        pltpu.sync_copy(zsil.at[pl.ds(base, 80)], zl_v)
        cd.wait()
        pltpu.sync_copy(zl_v, agg_sh.at[cols_v], add=True)
        return 0

    lax.fori_loop(0, EPT // 80, block, 0)

    plsc.subcore_barrier()
    pltpu.sync_copy(agg_sh.at[pl.ds(s * ROWS_PT, ROWS_PT)],
                    part.at[c, pl.ds(s * ROWS_PT, ROWS_PT)])


_MB = 2000


def _tc_proj(x, w0):
    def body(x_ref, w_ref, o_ref):
        o_ref[...] = jnp.dot(x_ref[...], w_ref[...],
                             preferred_element_type=jnp.float32)

    return pl.pallas_call(
        body,
        grid=(N // _MB,),
        in_specs=[pl.BlockSpec((_MB, D_IN), lambda i: (i, 0)),
                  pl.BlockSpec((D_IN, H), lambda i: (0, 0))],
        out_specs=pl.BlockSpec((_MB, H), lambda i: (i, 0)),
        out_shape=jax.ShapeDtypeStruct((N, H), jnp.float32),
    )(x, w0)


def _tc_pre(h, wa, wb):
    def body(h_ref, wa_ref, wb_ref, a_ref, b_ref):
        hv = h_ref[...]
        a_ref[...] = jnp.dot(hv, wa_ref[...], preferred_element_type=jnp.float32)
        b_ref[...] = jnp.dot(hv, wb_ref[...], preferred_element_type=jnp.float32)

    return pl.pallas_call(
        body,
        grid=(N // _MB,),
        in_specs=[pl.BlockSpec((_MB, H), lambda i: (i, 0)),
                  pl.BlockSpec((H, HP), lambda i: (0, 0)),
                  pl.BlockSpec((H, HP), lambda i: (0, 0))],
        out_specs=[pl.BlockSpec((_MB, HP), lambda i: (i, 0)),
                   pl.BlockSpec((_MB, HP), lambda i: (i, 0))],
        out_shape=[jax.ShapeDtypeStruct((N, HP), jnp.float32),
                   jax.ShapeDtypeStruct((N, HP), jnp.float32)],
    )(h, wa, wb)


def _tc_post(parts, h, wm, wc, wd, wu):
    def body(p0_ref, p1_ref, p2_ref, p3_ref, h_ref, wm_ref, wc_ref, wd_ref,
             wu_ref, o_ref):
        agg = jnp.zeros((_MB, H), jnp.float32)
        for j, p_ref in enumerate((p0_ref, p1_ref, p2_ref, p3_ref)):
            pj = p_ref[0] + p_ref[1]
            agg = agg + jnp.dot(pj, wm_ref[pl.ds(j * HC, HC), :],
                                preferred_element_type=jnp.float32)
        hv = h_ref[...]
        u = (jnp.dot(hv, wc_ref[...], preferred_element_type=jnp.float32)
             * jnp.dot(agg, wd_ref[...], preferred_element_type=jnp.float32))
        o_ref[...] = jnp.dot(_silu(u), wu_ref[...],
                             preferred_element_type=jnp.float32)

    pspec = pl.BlockSpec((2, _MB, HC), lambda i: (0, i, 0))
    wspec = pl.BlockSpec((H, H), lambda i: (0, 0))
    return pl.pallas_call(
        body,
        grid=(N // _MB,),
        in_specs=[pspec, pspec, pspec, pspec,
                  pl.BlockSpec((_MB, H), lambda i: (i, 0)),
                  pl.BlockSpec((HP, H), lambda i: (0, 0)),
                  wspec, wspec, wspec],
        out_specs=pl.BlockSpec((_MB, H), lambda i: (i, 0)),
        out_shape=jax.ShapeDtypeStruct((N, H), jnp.float32),
    )(*parts, h, wm, wc, wd, wu)


def _tc_silu(z):
    def body(z_ref, o_ref):
        o_ref[...] = jax.nn.silu(z_ref[...])

    return pl.pallas_call(
        body,
        grid=(40,),
        in_specs=[pl.BlockSpec((E // 40, HC), lambda i: (i, 0))],
        out_specs=pl.BlockSpec((E // 40, HC), lambda i: (i, 0)),
        out_shape=jax.ShapeDtypeStruct((E, HC), jnp.float32),
    )(z)


def _tc_norm(rx, ry, rz, d2):
    def body(rx_ref, ry_ref, rz_ref, d2_ref, nx_ref, ny_ref, nz_ref, dd_ref):
        dist = jnp.sqrt(d2_ref[...])
        inv = 1.0 / (dist + 1e-08)
        nx_ref[...] = rx_ref[...] * inv
        ny_ref[...] = ry_ref[...] * inv
        nz_ref[...] = rz_ref[...] * inv
        dd_ref[...] = dist

    shp = jax.ShapeDtypeStruct((E // 128, 128), jnp.float32)
    return pl.pallas_call(body, out_shape=[shp] * 4)(rx, ry, rz, d2)


def _tc_pool(h, batch2d, wout):
    def body(h_ref, b_ref, w_ref, o_ref, sum_acc, cnt_acc):
        i = pl.program_id(0)

        @pl.when(i == 0)
        def _():
            sum_acc[...] = jnp.zeros((B, S0), jnp.float32)
            cnt_acc[...] = jnp.zeros((1, B), jnp.float32)

        inv = h_ref[:, pl.ds(0, S0)]
        oh = (b_ref[...] == lax.broadcasted_iota(jnp.int32, (_MB, B), 1)
              ).astype(jnp.float32)
        sum_acc[...] += lax.dot_general(oh, inv, (((0,), (0,)), ((), ())),
                                        preferred_element_type=jnp.float32)
        cnt_acc[...] += jnp.sum(oh, axis=0, keepdims=True)

        @pl.when(i == N // _MB - 1)
        def _():
            pooled = sum_acc[...] / jnp.maximum(cnt_acc[...], 1.0).reshape(B, 1)
            o_ref[...] = jnp.dot(pooled, w_ref[...],
                                 preferred_element_type=jnp.float32)

    return pl.pallas_call(
        body,
        grid=(N // _MB,),
        in_specs=[pl.BlockSpec((_MB, H), lambda i: (i, 0)),
                  pl.BlockSpec((_MB, 1), lambda i: (i, 0)),
                  pl.BlockSpec((S0, D_OUT), lambda i: (0, 0))],
        out_specs=pl.BlockSpec((B, D_OUT), lambda i: (0, 0)),
        out_shape=jax.ShapeDtypeStruct((B, D_OUT), jnp.float32),
        scratch_shapes=[pltpu.VMEM((B, S0), jnp.float32),
                        pltpu.VMEM((1, B), jnp.float32)],
    )(h, batch2d, wout)


def kernel(x, pos, edge_index, batch, W0, Wa, Wb, We, Wm, Wc, Wd, Wu, Wout):
    row, col = edge_index[0], edge_index[1]

    rx, ry, rz, d2 = _sc_geom(pos[:, 0], pos[:, 1], pos[:, 2], row, col)
    nx, ny, nz, dd = _tc_norm(rx.reshape(E // 128, 128),
                              ry.reshape(E // 128, 128),
                              rz.reshape(E // 128, 128),
                              d2.reshape(E // 128, 128))
    ea_flat = jnp.stack([nx.reshape(E), ny.reshape(E), nz.reshape(E),
                         dd.reshape(E)], axis=1).reshape(4 * E)

    pad = ((0, 0), (0, 0), (0, HP - H))
    Wa_p = jnp.pad(Wa, pad)
    Wb_p = jnp.pad(Wb, pad)
    We_p = jnp.pad(We, pad)
    Wm_p = jnp.pad(Wm, ((0, 0), (0, HP - H), (0, 0)))

    h = _tc_proj(x, W0)
    for l in range(L):
        ha, hb = _tc_pre(h, Wa_p[l], Wb_p[l])
        parts = []
        for j in range(NCHUNK):
            sl = slice(j * HC, (j + 1) * HC)
            zr = _sc_mul(ha[:, sl], hb[:, sl], ea_flat, row, col,
                         We_p[l][:, sl])
            parts.append(_sc_scat(_tc_silu(zr), col))
        h = _tc_post(parts, h, Wm_p[l], Wc[l], Wd[l], Wu[l])

    return _tc_pool(h, batch.reshape(N, 1), Wout)
